# baseline (speedup 1.0000x reference)
import functools

import jax
import jax.numpy as jnp
from jax import lax
from jax.experimental import pallas as pl
from jax.experimental.pallas import tpu as pltpu
from jax.experimental.pallas import tpu_sc as plsc

WIDTH = 16
N_NODES = 10000
N_EDGES = 320000
D_EDGE = 16

NC = 2
NS = 16
NW = NC * NS
EW = N_EDGES // NW
ROW = 125
RPW = EW // ROW
CH_ROWS = 8
CH_E = CH_ROWS * ROW
NCHUNK = RPW // CH_ROWS
NPT = N_NODES // NS

EB = 6400
HI = lax.Precision.HIGHEST


def _selT():
    r = lax.broadcasted_iota(jnp.int32, (WIDTH, WIDTH * WIDTH), 0)
    c = lax.broadcasted_iota(jnp.int32, (WIDTH, WIDTH * WIDTH), 1)
    return jnp.where(c // WIDTH == r, 1.0, 0.0)


def _wgen_body(at_ref, w_ref, b_ref, o_ref):
    aT = at_ref[...]
    W = w_ref[...]
    b2 = b_ref[...]
    w = lax.dot_general(aT, W, (((0,), (0,)), ((), ())),
                        preferred_element_type=jnp.float32)
    wb = (w + b2).astype(jnp.bfloat16).astype(jnp.float32)
    o_ref[...] = lax.dot_general(_selT(), wb, (((1,), (1,)), ((), ())),
                                 preferred_element_type=jnp.float32,
                                 precision=HI)


_wgen = pl.pallas_call(
    _wgen_body,
    grid=(N_EDGES // EB,),
    in_specs=[
        pl.BlockSpec((WIDTH, EB), lambda b: (0, b)),
        pl.BlockSpec((WIDTH, WIDTH * WIDTH), lambda b: (0, 0)),
        pl.BlockSpec((1, WIDTH * WIDTH), lambda b: (0, 0)),
    ],
    out_specs=pl.BlockSpec((WIDTH, EB), lambda b: (0, b)),
    out_shape=jax.ShapeDtypeStruct((WIDTH, N_EDGES), jnp.float32),
)


def _sc_segsum_body(idx_hbm, wT_hbm, s_out,
                    idx_buf, tbuf, vbuf, stage, s_sh,
                    sem_idx, sem_t0, sem_t1, sem_sc):
    cid = lax.axis_index("c")
    sid = lax.axis_index("s")
    wid = sid * NC + cid

    idx_cp = pltpu.async_copy(idx_hbm.at[wid], idx_buf, sem_idx)
    t_sems = (sem_t0, sem_t1)
    e_base = wid * EW
    pltpu.async_copy(wT_hbm.at[:, pl.ds(e_base, CH_E)], tbuf.at[0], sem_t0)
    pltpu.async_copy(wT_hbm.at[:, pl.ds(e_base + CH_E, CH_E)],
                     tbuf.at[1], sem_t1)

    def zfill(i, _):
        stage[i, :] = jnp.zeros((16,), jnp.float32)
        return 0
    lax.fori_loop(0, NPT, zfill, 0)

    pltpu.sync_copy(stage, s_sh.at[pl.ds(sid * NPT, NPT)])
    idx_cp.wait()
    plsc.subcore_barrier()

    rows16 = jax.lax.iota(jnp.int32, 16)

    def do_chunk(k, par):
        pltpu.make_async_copy(wT_hbm.at[:, pl.ds(e_base, CH_E)],
                              tbuf.at[par], t_sems[par]).wait()

        def repack(e, _):
            cols = jnp.zeros((16,), jnp.int32) + e
            vbuf[e, :] = plsc.load_gather(tbuf.at[par], [rows16, cols])
            return 0
        lax.fori_loop(0, CH_E, repack, 0)

        @pl.when(k + 2 < NCHUNK)
        def _():
            pltpu.async_copy(
                wT_hbm.at[:, pl.ds(e_base + (k + 2) * CH_E, CH_E)],
                tbuf.at[par], t_sems[par])

        for j in range(CH_ROWS):
            r = k * CH_ROWS + j
            pltpu.async_copy(vbuf.at[pl.ds(j * ROW, ROW)],
                             s_sh.at[idx_buf.at[r]], sem_sc, add=True)
        for j in range(CH_ROWS):
            r = k * CH_ROWS + j
            pltpu.make_async_copy(vbuf.at[pl.ds(j * ROW, ROW)],
                                  s_sh.at[idx_buf.at[r]], sem_sc).wait()

    def chunk_pair(t, _):
        do_chunk(2 * t, 0)
        do_chunk(2 * t + 1, 1)
        return 0
    lax.fori_loop(0, NCHUNK // 2, chunk_pair, 0)

    plsc.subcore_barrier()

    pltpu.sync_copy(s_sh.at[pl.ds(sid * NPT, NPT)], stage)
    pltpu.sync_copy(stage, s_out.at[cid, pl.ds(sid * NPT, NPT)])


_sc_segsum = pl.kernel(
    _sc_segsum_body,
    out_type=jax.ShapeDtypeStruct((NC, N_NODES, D_EDGE), jnp.float32),
    mesh=plsc.VectorSubcoreMesh(
        core_axis_name="c", subcore_axis_name="s",
        num_cores=NC, num_subcores=NS),
    compiler_params=pltpu.CompilerParams(use_tc_tiling_on_sc=False,
                                         needs_layout_passes=False),
    scratch_types=[
        pltpu.VMEM((RPW, ROW), jnp.int32),
        pltpu.VMEM((2, D_EDGE, CH_E), jnp.float32),
        pltpu.VMEM((CH_E, D_EDGE), jnp.float32),
        pltpu.VMEM((NPT, D_EDGE), jnp.float32),
        pltpu.VMEM_SHARED((N_NODES, D_EDGE), jnp.float32),
        pltpu.SemaphoreType.DMA,
        pltpu.SemaphoreType.DMA,
        pltpu.SemaphoreType.DMA,
        pltpu.SemaphoreType.DMA,
    ],
)


def _finish_body(xt_ref, s_ref, o_ref):
    xtb = xt_ref[...].astype(jnp.bfloat16).astype(jnp.float32)
    S = s_ref[0] + s_ref[1]
    M = lax.dot_general(xtb, S, (((1,), (0,)), ((), ())),
                        preferred_element_type=jnp.float32,
                        precision=HI)
    r = lax.broadcasted_iota(jnp.int32, (WIDTH, WIDTH), 0)
    c = lax.broadcasted_iota(jnp.int32, (WIDTH, WIDTH), 1)
    trace = jnp.sum(jnp.where(r == c, M, 0.0))
    o_ref[0, 0] = trace * (1.0 / (N_EDGES * WIDTH))


_finish = pl.pallas_call(
    _finish_body,
    out_shape=jax.ShapeDtypeStruct((1, 1), jnp.float32),
    out_specs=pl.BlockSpec(memory_space=pltpu.SMEM),
)


def kernel(x, edge_index, edge_attr, W_nn, b_nn):
    idx3d = edge_index[1].reshape(NW, RPW, ROW)
    wrowT = _wgen(edge_attr.T, W_nn, b_nn.reshape(1, WIDTH * WIDTH))
    sw2 = _sc_segsum(idx3d, wrowT)
    out = _finish(x.T, sw2)
    return out[0, 0]


# speedup vs baseline: 1.9745x; 1.9745x over previous
import functools

import jax
import jax.numpy as jnp
from jax import lax
from jax.experimental import pallas as pl
from jax.experimental.pallas import tpu as pltpu
from jax.experimental.pallas import tpu_sc as plsc

WIDTH = 16
N_NODES = 10000
N_EDGES = 320000
D_EDGE = 16

NC = 2
NS = 16
NW = NC * NS
EW = N_EDGES // NW
ROW = 125
RPW = EW // ROW
CH_ROWS = 8
CH_E = CH_ROWS * ROW
NCHUNK = RPW // CH_ROWS
NPT = N_NODES // NS

EB = 12800
HI = lax.Precision.HIGHEST


def _selT():
    r = lax.broadcasted_iota(jnp.int32, (WIDTH, WIDTH * WIDTH), 0)
    c = lax.broadcasted_iota(jnp.int32, (WIDTH, WIDTH * WIDTH), 1)
    return jnp.where(c // WIDTH == r, 1.0, 0.0)


def _wgen_body(at_ref, w_ref, b_ref, o_ref):
    aT = at_ref[...]
    W = w_ref[...]
    b2 = b_ref[...]
    w = lax.dot_general(aT, W, (((0,), (0,)), ((), ())),
                        preferred_element_type=jnp.float32)
    wb = (w + b2).astype(jnp.bfloat16)
    o_ref[...] = lax.dot_general(_selT().astype(jnp.bfloat16), wb,
                                 (((1,), (1,)), ((), ())),
                                 preferred_element_type=jnp.float32)


_wgen = pl.pallas_call(
    _wgen_body,
    grid=(N_EDGES // EB,),
    in_specs=[
        pl.BlockSpec((WIDTH, EB), lambda b: (0, b)),
        pl.BlockSpec((WIDTH, WIDTH * WIDTH), lambda b: (0, 0)),
        pl.BlockSpec((1, WIDTH * WIDTH), lambda b: (0, 0)),
    ],
    out_specs=pl.BlockSpec((WIDTH, EB), lambda b: (0, b)),
    out_shape=jax.ShapeDtypeStruct((WIDTH, N_EDGES), jnp.float32),
)


def _sc_segsum_body(idx_hbm, wT_hbm, s_out,
                    idx_buf, tbuf, vbuf, stage, s_sh,
                    sem_idx, sem_t0, sem_t1, sem_sc):
    cid = lax.axis_index("c")
    sid = lax.axis_index("s")
    wid = sid * NC + cid

    idx_cp = pltpu.async_copy(idx_hbm.at[wid], idx_buf, sem_idx)
    t_sems = (sem_t0, sem_t1)
    e_base = wid * EW
    pltpu.async_copy(wT_hbm.at[:, pl.ds(e_base, CH_E)], tbuf.at[0], sem_t0)
    pltpu.async_copy(wT_hbm.at[:, pl.ds(e_base + CH_E, CH_E)],
                     tbuf.at[1], sem_t1)

    def zfill(i, _):
        stage[i, :] = jnp.zeros((16,), jnp.float32)
        return 0
    lax.fori_loop(0, NPT, zfill, 0)

    pltpu.sync_copy(stage, s_sh.at[pl.ds(sid * NPT, NPT)])
    idx_cp.wait()
    plsc.subcore_barrier()

    rows16 = jax.lax.iota(jnp.int32, 16)

    def do_chunk(k, par):
        pltpu.make_async_copy(wT_hbm.at[:, pl.ds(e_base, CH_E)],
                              tbuf.at[par], t_sems[par]).wait()

        def repack(e, _):
            cols = jnp.zeros((16,), jnp.int32) + e
            vbuf[e, :] = plsc.load_gather(tbuf.at[par], [rows16, cols])
            return 0
        lax.fori_loop(0, CH_E, repack, 0)

        @pl.when(k + 2 < NCHUNK)
        def _():
            pltpu.async_copy(
                wT_hbm.at[:, pl.ds(e_base + (k + 2) * CH_E, CH_E)],
                tbuf.at[par], t_sems[par])

        for j in range(CH_ROWS):
            r = k * CH_ROWS + j
            pltpu.async_copy(vbuf.at[pl.ds(j * ROW, ROW)],
                             s_sh.at[idx_buf.at[r]], sem_sc, add=True)
        for j in range(CH_ROWS):
            r = k * CH_ROWS + j
            pltpu.make_async_copy(vbuf.at[pl.ds(j * ROW, ROW)],
                                  s_sh.at[idx_buf.at[r]], sem_sc).wait()

    def chunk_pair(t, _):
        do_chunk(2 * t, 0)
        do_chunk(2 * t + 1, 1)
        return 0
    lax.fori_loop(0, NCHUNK // 2, chunk_pair, 0)

    plsc.subcore_barrier()

    pltpu.sync_copy(s_sh.at[pl.ds(sid * NPT, NPT)], stage)
    pltpu.sync_copy(stage, s_out.at[cid, pl.ds(sid * NPT, NPT)])


_sc_segsum = pl.kernel(
    _sc_segsum_body,
    out_type=jax.ShapeDtypeStruct((NC, N_NODES, D_EDGE), jnp.float32),
    mesh=plsc.VectorSubcoreMesh(
        core_axis_name="c", subcore_axis_name="s",
        num_cores=NC, num_subcores=NS),
    compiler_params=pltpu.CompilerParams(use_tc_tiling_on_sc=False,
                                         needs_layout_passes=False),
    scratch_types=[
        pltpu.VMEM((RPW, ROW), jnp.int32),
        pltpu.VMEM((2, D_EDGE, CH_E), jnp.float32),
        pltpu.VMEM((CH_E, D_EDGE), jnp.float32),
        pltpu.VMEM((NPT, D_EDGE), jnp.float32),
        pltpu.VMEM_SHARED((N_NODES, D_EDGE), jnp.float32),
        pltpu.SemaphoreType.DMA,
        pltpu.SemaphoreType.DMA,
        pltpu.SemaphoreType.DMA,
        pltpu.SemaphoreType.DMA,
    ],
)


def _finish_body(xt_ref, s_ref, o_ref):
    xtb = xt_ref[...].astype(jnp.bfloat16).astype(jnp.float32)
    S = s_ref[0] + s_ref[1]
    M = lax.dot_general(xtb, S, (((1,), (0,)), ((), ())),
                        preferred_element_type=jnp.float32,
                        precision=HI)
    r = lax.broadcasted_iota(jnp.int32, (WIDTH, WIDTH), 0)
    c = lax.broadcasted_iota(jnp.int32, (WIDTH, WIDTH), 1)
    trace = jnp.sum(jnp.where(r == c, M, 0.0))
    o_ref[0, 0] = trace * (1.0 / (N_EDGES * WIDTH))


_finish = pl.pallas_call(
    _finish_body,
    out_shape=jax.ShapeDtypeStruct((1, 1), jnp.float32),
    out_specs=pl.BlockSpec(memory_space=pltpu.SMEM),
)


def kernel(x, edge_index, edge_attr, W_nn, b_nn):
    idx3d = edge_index[1].reshape(NW, RPW, ROW)
    wrowT = _wgen(edge_attr.T, W_nn, b_nn.reshape(1, WIDTH * WIDTH))
    sw2 = _sc_segsum(idx3d, wrowT)
    out = _finish(x.T, sw2)
    return out[0, 0]
